# Initial kernel scaffold; baseline (speedup 1.0000x reference)
import jax
import jax.numpy as jnp
from jax.experimental import pallas as pl


def kernel(x, Wr, br, Wg, Wu, Wd, Wsg, Wsu, Wsd):
    raise NotImplementedError("write your pallas kernel here")


# speedup vs baseline: 1.0098x; 1.0098x over previous
import functools

import jax
import jax.numpy as jnp
from jax import lax
from jax.experimental import pallas as pl
from jax.experimental.pallas import tpu as pltpu
from jax.experimental.pallas import tpu_sc as plsc

_T, _D, _E, _K, _FF = 2048, 768, 8, 2, 2048
_TM = 128
_PP = _T * _K + _E * _TM
_NT = _PP // _TM
_NEG = -1e30


def _router_body(x_ref, wr_ref, br_ref, out_ref):
    l = jnp.dot(x_ref[...], wr_ref[...],
                preferred_element_type=jnp.float32) + br_ref[...]
    m = jnp.max(l, axis=1, keepdims=True)
    p = jnp.exp(l - m)
    lanes = lax.broadcasted_iota(jnp.int32, p.shape, 1)
    a1 = jnp.argmax(p, axis=1)[:, None]
    p1 = jnp.max(p, axis=1, keepdims=True)
    pm = jnp.where(lanes == a1, -1.0, p)
    a2 = jnp.argmax(pm, axis=1)[:, None]
    p2 = jnp.max(pm, axis=1, keepdims=True)
    s = p1 + p2 + 1e-20
    out_ref[...] = jnp.where(
        lanes == 0, a1.astype(jnp.float32),
        jnp.where(lanes == 1, a2.astype(jnp.float32),
                  jnp.where(lanes == 2, p1 / s,
                            jnp.where(lanes == 3, p2 / s, 0.0))))


def _router(flat, wr_pad, br_pad):
    return pl.pallas_call(
        _router_body,
        out_shape=jax.ShapeDtypeStruct((_T, 128), jnp.float32),
    )(flat, wr_pad, br_pad)


def _sc_gather(table, idx):
    rows, dd = table.shape
    (batch,) = idx.shape
    info = plsc.get_sparse_core_info()
    nw = info.num_cores * info.num_subcores
    assert batch % (8 * nw) == 0 and dd % info.num_lanes == 0
    bw = batch // nw
    mesh = plsc.VectorSubcoreMesh(core_axis_name="c", subcore_axis_name="s")

    @functools.partial(
        pl.kernel, mesh=mesh,
        out_type=jax.ShapeDtypeStruct((batch, dd), table.dtype),
        scratch_types=[
            pltpu.VMEM((bw,), jnp.int32),
            pltpu.VMEM((bw, dd), table.dtype),
            pltpu.SemaphoreType.DMA,
        ],
    )
    def k(table_hbm, idx_hbm, out_hbm, idx_v, rows_v, sem):
        wid = lax.axis_index("s") * info.num_cores + lax.axis_index("c")
        base = wid * bw
        pltpu.sync_copy(idx_hbm.at[pl.ds(base, bw)], idx_v)
        pltpu.async_copy(table_hbm.at[idx_v], rows_v, sem).wait()
        pltpu.sync_copy(rows_v, out_hbm.at[pl.ds(base, bw)])

    return k(table, idx)


def _ffn_body(te_ref, xs_ref, wg_ref, wu_ref, wd_ref, w_ref, out_ref):
    xb = xs_ref[...].astype(jnp.bfloat16)
    g = jnp.dot(xb, wg_ref[0], preferred_element_type=jnp.float32)
    u = jnp.dot(xb, wu_ref[0], preferred_element_type=jnp.float32)
    h = g * jax.nn.sigmoid(g) * u * w_ref[...]
    out_ref[...] = jnp.dot(h.astype(jnp.bfloat16), wd_ref[0],
                           preferred_element_type=jnp.float32)


def _ffn_grouped(tile_expert, xs, wg, wu, wd, w_col):
    grid_spec = pltpu.PrefetchScalarGridSpec(
        num_scalar_prefetch=1,
        grid=(_NT,),
        in_specs=[
            pl.BlockSpec((_TM, _D), lambda i, te: (i, 0)),
            pl.BlockSpec((1, _D, _FF), lambda i, te: (te[i], 0, 0)),
            pl.BlockSpec((1, _D, _FF), lambda i, te: (te[i], 0, 0)),
            pl.BlockSpec((1, _FF, _D), lambda i, te: (te[i], 0, 0)),
            pl.BlockSpec((_TM, 1), lambda i, te: (i, 0)),
        ],
        out_specs=pl.BlockSpec((_TM, _D), lambda i, te: (i, 0)),
    )
    return pl.pallas_call(
        _ffn_body, grid_spec=grid_spec,
        out_shape=jax.ShapeDtypeStruct((_PP, _D), jnp.float32),
    )(tile_expert, xs, wg, wu, wd, w_col)


def _shared_body(x_ref, wg_ref, wu_ref, wd_ref, yp_ref, out_ref):
    xb = x_ref[...].astype(jnp.bfloat16)
    g = jnp.dot(xb, wg_ref[...], preferred_element_type=jnp.float32)
    u = jnp.dot(xb, wu_ref[...], preferred_element_type=jnp.float32)
    h = g * jax.nn.sigmoid(g) * u
    y = jnp.dot(h.astype(jnp.bfloat16), wd_ref[...],
                preferred_element_type=jnp.float32)
    out_ref[...] = y + yp_ref[:, :_D] + yp_ref[:, _D:]


def _shared(flat, wsg, wsu, wsd, yp2):
    bt = 256
    return pl.pallas_call(
        _shared_body,
        grid=(_T // bt,),
        in_specs=[
            pl.BlockSpec((bt, _D), lambda i: (i, 0)),
            pl.BlockSpec((_D, _FF), lambda i: (0, 0)),
            pl.BlockSpec((_D, _FF), lambda i: (0, 0)),
            pl.BlockSpec((_FF, _D), lambda i: (0, 0)),
            pl.BlockSpec((bt, _K * _D), lambda i: (i, 0)),
        ],
        out_specs=pl.BlockSpec((bt, _D), lambda i: (i, 0)),
        out_shape=jax.ShapeDtypeStruct((_T, _D), jnp.float32),
    )(flat, wsg, wsu, wsd, yp2)


def _routing_metadata(r):
    ids = r[:, :_K].astype(jnp.int32)
    w = r[:, _K:2 * _K]
    e_flat = ids.reshape(-1)
    onehot = (e_flat[:, None] == jnp.arange(_E, dtype=jnp.int32)[None, :])
    csum = jnp.cumsum(onehot.astype(jnp.int32), axis=0)
    rank = jnp.take_along_axis(csum, e_flat[:, None], axis=1)[:, 0] - 1
    counts = csum[-1]
    pcounts = ((counts + _TM - 1) // _TM) * _TM
    off = jnp.concatenate(
        [jnp.zeros((1,), jnp.int32), jnp.cumsum(pcounts).astype(jnp.int32)])
    dest = off[e_flat] + rank
    tok = jnp.arange(_T * _K, dtype=jnp.int32) // _K
    sorted_tok = jnp.zeros((_PP,), jnp.int32).at[dest].set(tok)
    w_sorted = jnp.zeros((_PP,), jnp.float32).at[dest].set(w.reshape(-1))
    tile_expert = jnp.clip(
        jnp.searchsorted(off, jnp.arange(_NT, dtype=jnp.int32) * _TM,
                         side="right") - 1,
        0, _E - 1).astype(jnp.int32)
    return dest, sorted_tok, w_sorted, tile_expert


def kernel(x, Wr, br, Wg, Wu, Wd, Wsg, Wsu, Wsd):
    b, s, d = x.shape
    flat = x.reshape(-1, d)
    wr_pad = jnp.zeros((_D, 128), jnp.float32).at[:, :_E].set(Wr)
    br_pad = jnp.full((1, 128), _NEG, jnp.float32).at[0, :_E].set(br)
    r = _router(flat, wr_pad, br_pad)
    dest, sorted_tok, w_sorted, tile_expert = _routing_metadata(r)

    xs = _sc_gather(flat, sorted_tok)
    ys = _ffn_grouped(tile_expert, xs,
                      Wg.astype(jnp.bfloat16), Wu.astype(jnp.bfloat16),
                      Wd.astype(jnp.bfloat16), w_sorted.reshape(_PP, 1))
    yp = _sc_gather(ys, dest)
    yp2 = yp.reshape(_T, _K * _D)
    y = _shared(flat, Wsg.astype(jnp.bfloat16), Wsu.astype(jnp.bfloat16),
                Wsd.astype(jnp.bfloat16), yp2)
    return y.reshape(b, s, d)


# speedup vs baseline: 1.1909x; 1.1794x over previous
import functools

import jax
import jax.numpy as jnp
from jax import lax
from jax.experimental import pallas as pl
from jax.experimental.pallas import tpu as pltpu
from jax.experimental.pallas import tpu_sc as plsc

_T, _D, _E, _K, _FF = 2048, 768, 8, 2, 2048
_TM = 128
_PP = _T * _K + _E * _TM
_NT = _PP // _TM
_NEG = -1e30


def _router_body(x_ref, wr_ref, br_ref, out_ref):
    l = jnp.dot(x_ref[...], wr_ref[...],
                preferred_element_type=jnp.float32) + br_ref[...]
    m = jnp.max(l, axis=1, keepdims=True)
    p = jnp.exp(l - m)
    lanes = lax.broadcasted_iota(jnp.int32, p.shape, 1)
    a1 = jnp.argmax(p, axis=1)[:, None]
    p1 = jnp.max(p, axis=1, keepdims=True)
    pm = jnp.where(lanes == a1, -1.0, p)
    a2 = jnp.argmax(pm, axis=1)[:, None]
    p2 = jnp.max(pm, axis=1, keepdims=True)
    s = p1 + p2 + 1e-20
    out_ref[...] = jnp.where(
        lanes == 0, a1.astype(jnp.float32),
        jnp.where(lanes == 1, a2.astype(jnp.float32),
                  jnp.where(lanes == 2, p1 / s,
                            jnp.where(lanes == 3, p2 / s, 0.0))))


def _router(flat, wr_pad, br_pad):
    return pl.pallas_call(
        _router_body,
        out_shape=jax.ShapeDtypeStruct((_T, 128), jnp.float32),
    )(flat, wr_pad, br_pad)


def _cast_body(wg_ref, wu_ref, wd_ref, og_ref, ou_ref, od_ref):
    og_ref[...] = wg_ref[...].astype(jnp.bfloat16)
    ou_ref[...] = wu_ref[...].astype(jnp.bfloat16)
    od_ref[...] = wd_ref[...].astype(jnp.bfloat16)


def _cast_experts(wg, wu, wd):
    return pl.pallas_call(
        _cast_body,
        grid=(_E,),
        in_specs=[
            pl.BlockSpec((1, _D, _FF), lambda i: (i, 0, 0)),
            pl.BlockSpec((1, _D, _FF), lambda i: (i, 0, 0)),
            pl.BlockSpec((1, _FF, _D), lambda i: (i, 0, 0)),
        ],
        out_specs=[
            pl.BlockSpec((1, _D, _FF), lambda i: (i, 0, 0)),
            pl.BlockSpec((1, _D, _FF), lambda i: (i, 0, 0)),
            pl.BlockSpec((1, _FF, _D), lambda i: (i, 0, 0)),
        ],
        out_shape=[
            jax.ShapeDtypeStruct((_E, _D, _FF), jnp.bfloat16),
            jax.ShapeDtypeStruct((_E, _D, _FF), jnp.bfloat16),
            jax.ShapeDtypeStruct((_E, _FF, _D), jnp.bfloat16),
        ],
    )(wg, wu, wd)


def _cast4_body(a_ref, b_ref, c_ref, d_ref, oa_ref, ob_ref, oc_ref, od_ref):
    oa_ref[...] = a_ref[...].astype(jnp.bfloat16)
    ob_ref[...] = b_ref[...].astype(jnp.bfloat16)
    oc_ref[...] = c_ref[...].astype(jnp.bfloat16)
    od_ref[...] = d_ref[...].astype(jnp.bfloat16)


def _cast_shared(flat, wsg, wsu, wsd):
    return pl.pallas_call(
        _cast4_body,
        out_shape=[
            jax.ShapeDtypeStruct(flat.shape, jnp.bfloat16),
            jax.ShapeDtypeStruct(wsg.shape, jnp.bfloat16),
            jax.ShapeDtypeStruct(wsu.shape, jnp.bfloat16),
            jax.ShapeDtypeStruct(wsd.shape, jnp.bfloat16),
        ],
    )(flat, wsg, wsu, wsd)


def _ffn_body(te_ref, tok_ref, tbl_ref, wg_ref, wu_ref, wd_ref, w_ref,
              out_ref):
    idx = tok_ref[0]
    oh = (idx == lax.broadcasted_iota(jnp.int32, (_TM, _T), 1))
    xb = jnp.dot(oh.astype(jnp.bfloat16), tbl_ref[...],
                 preferred_element_type=jnp.float32).astype(jnp.bfloat16)
    g = jnp.dot(xb, wg_ref[0], preferred_element_type=jnp.float32)
    u = jnp.dot(xb, wu_ref[0], preferred_element_type=jnp.float32)
    h = g * jax.nn.sigmoid(g) * u * w_ref[...]
    out_ref[...] = jnp.dot(h.astype(jnp.bfloat16), wd_ref[0],
                           preferred_element_type=jnp.float32)


def _ffn_grouped(tile_expert, tok_col, tbl16, wg, wu, wd, w_col):
    grid_spec = pltpu.PrefetchScalarGridSpec(
        num_scalar_prefetch=1,
        grid=(_NT,),
        in_specs=[
            pl.BlockSpec((1, _TM, 1), lambda i, te: (i, 0, 0)),
            pl.BlockSpec((_T, _D), lambda i, te: (0, 0)),
            pl.BlockSpec((1, _D, _FF), lambda i, te: (te[i], 0, 0)),
            pl.BlockSpec((1, _D, _FF), lambda i, te: (te[i], 0, 0)),
            pl.BlockSpec((1, _FF, _D), lambda i, te: (te[i], 0, 0)),
            pl.BlockSpec((_TM, 1), lambda i, te: (i, 0)),
        ],
        out_specs=pl.BlockSpec((_TM, _D), lambda i, te: (i, 0)),
    )
    return pl.pallas_call(
        _ffn_body, grid_spec=grid_spec,
        out_shape=jax.ShapeDtypeStruct((_PP, _D), jnp.float32),
    )(tile_expert, tok_col, tbl16, wg, wu, wd, w_col)


def _sc_gather(table, idx):
    rows, dd = table.shape
    (batch,) = idx.shape
    info = plsc.get_sparse_core_info()
    nw = info.num_cores * info.num_subcores
    assert batch % (8 * nw) == 0 and dd % info.num_lanes == 0
    bw = batch // nw
    mesh = plsc.VectorSubcoreMesh(core_axis_name="c", subcore_axis_name="s")

    @functools.partial(
        pl.kernel, mesh=mesh,
        out_type=jax.ShapeDtypeStruct((batch, dd), table.dtype),
        scratch_types=[
            pltpu.VMEM((bw,), jnp.int32),
            pltpu.VMEM((bw, dd), table.dtype),
            pltpu.SemaphoreType.DMA,
        ],
    )
    def k(table_hbm, idx_hbm, out_hbm, idx_v, rows_v, sem):
        wid = lax.axis_index("s") * info.num_cores + lax.axis_index("c")
        base = wid * bw
        pltpu.sync_copy(idx_hbm.at[pl.ds(base, bw)], idx_v)
        pltpu.async_copy(table_hbm.at[idx_v], rows_v, sem).wait()
        pltpu.sync_copy(rows_v, out_hbm.at[pl.ds(base, bw)])

    return k(table, idx)


def _shared_body(x_ref, wg_ref, wu_ref, wd_ref, yp0_ref, yp1_ref, out_ref):
    xb = x_ref[...]
    g = jnp.dot(xb, wg_ref[...], preferred_element_type=jnp.float32)
    u = jnp.dot(xb, wu_ref[...], preferred_element_type=jnp.float32)
    h = g * jax.nn.sigmoid(g) * u
    y = jnp.dot(h.astype(jnp.bfloat16), wd_ref[...],
                preferred_element_type=jnp.float32)
    out_ref[...] = y + yp0_ref[...] + yp1_ref[...]


def _shared(flat16, wsg, wsu, wsd, yp):
    bt = 256
    nb = _T // bt
    return pl.pallas_call(
        _shared_body,
        grid=(nb,),
        in_specs=[
            pl.BlockSpec((bt, _D), lambda i: (i, 0)),
            pl.BlockSpec((_D, _FF), lambda i: (0, 0)),
            pl.BlockSpec((_D, _FF), lambda i: (0, 0)),
            pl.BlockSpec((_FF, _D), lambda i: (0, 0)),
            pl.BlockSpec((bt, _D), lambda i: (i, 0)),
            pl.BlockSpec((bt, _D), lambda i, _nb=nb: (i + _nb, 0)),
        ],
        out_specs=pl.BlockSpec((bt, _D), lambda i: (i, 0)),
        out_shape=jax.ShapeDtypeStruct((_T, _D), jnp.float32),
    )(flat16, wsg, wsu, wsd, yp, yp)


def _routing_metadata(r):
    ids = r[:, :_K].astype(jnp.int32)
    w = r[:, _K:2 * _K]
    e_flat = ids.reshape(-1)
    onehot = (e_flat[:, None] == jnp.arange(_E, dtype=jnp.int32)[None, :])
    oh32 = onehot.astype(jnp.int32)
    csum = jnp.cumsum(oh32, axis=0)
    rank = jnp.sum(csum * oh32, axis=1) - 1
    counts = csum[-1]
    pcounts = ((counts + _TM - 1) // _TM) * _TM
    off = jnp.concatenate(
        [jnp.zeros((1,), jnp.int32), jnp.cumsum(pcounts).astype(jnp.int32)])
    dest = jnp.sum(off[None, :_E] * oh32, axis=1) + rank
    tok = jnp.arange(_T * _K, dtype=jnp.int32) // _K
    sorted_tok = jnp.zeros((_PP,), jnp.int32).at[dest].set(tok)
    w_sorted = jnp.zeros((_PP,), jnp.float32).at[dest].set(w.reshape(-1))
    tile_expert = jnp.clip(
        jnp.searchsorted(off, jnp.arange(_NT, dtype=jnp.int32) * _TM,
                         side="right") - 1,
        0, _E - 1).astype(jnp.int32)
    dest_k = dest.reshape(_T, _K).T.reshape(-1)
    return dest_k, sorted_tok, w_sorted, tile_expert


def kernel(x, Wr, br, Wg, Wu, Wd, Wsg, Wsu, Wsd):
    b, s, d = x.shape
    flat = x.reshape(-1, d)
    wr_pad = jnp.zeros((_D, 128), jnp.float32).at[:, :_E].set(Wr)
    br_pad = jnp.full((1, 128), _NEG, jnp.float32).at[0, :_E].set(br)
    r = _router(flat, wr_pad, br_pad)
    dest_k, sorted_tok, w_sorted, tile_expert = _routing_metadata(r)

    wg16, wu16, wd16 = _cast_experts(Wg, Wu, Wd)
    flat16, wsg16, wsu16, wsd16 = _cast_shared(flat, Wsg, Wsu, Wsd)

    ys = _ffn_grouped(tile_expert, sorted_tok.reshape(_NT, _TM, 1), flat16,
                      wg16, wu16, wd16, w_sorted.reshape(_PP, 1))
    yp = _sc_gather(ys, dest_k)
    y = _shared(flat16, wsg16, wsu16, wsd16, yp)
    return y.reshape(b, s, d)


# speedup vs baseline: 1.2696x; 1.0661x over previous
import functools

import jax
import jax.numpy as jnp
from jax import lax
from jax.experimental import pallas as pl
from jax.experimental.pallas import tpu as pltpu
from jax.experimental.pallas import tpu_sc as plsc

_T, _D, _E, _K, _FF = 2048, 768, 8, 2, 2048
_TM = 256
_PP = _T * _K + _E * _TM
_NT = _PP // _TM
_NEG = -1e30


def _router_body(x_ref, wr_ref, br_ref, out_ref):
    l = jnp.dot(x_ref[...], wr_ref[...],
                preferred_element_type=jnp.float32) + br_ref[...]
    m = jnp.max(l, axis=1, keepdims=True)
    p = jnp.exp(l - m)
    lanes = lax.broadcasted_iota(jnp.int32, p.shape, 1)
    a1 = jnp.argmax(p, axis=1)[:, None]
    p1 = jnp.max(p, axis=1, keepdims=True)
    pm = jnp.where(lanes == a1, -1.0, p)
    a2 = jnp.argmax(pm, axis=1)[:, None]
    p2 = jnp.max(pm, axis=1, keepdims=True)
    s = p1 + p2 + 1e-20
    out_ref[...] = jnp.where(
        lanes == 0, a1.astype(jnp.float32),
        jnp.where(lanes == 1, a2.astype(jnp.float32),
                  jnp.where(lanes == 2, p1 / s,
                            jnp.where(lanes == 3, p2 / s, 0.0))))


def _router(flat, wr_pad, br_pad):
    return pl.pallas_call(
        _router_body,
        out_shape=jax.ShapeDtypeStruct((_T, 128), jnp.float32),
    )(flat, wr_pad, br_pad)


def _cast_body(wg_ref, wu_ref, wd_ref, og_ref, ou_ref, od_ref):
    og_ref[...] = wg_ref[...].astype(jnp.bfloat16)
    ou_ref[...] = wu_ref[...].astype(jnp.bfloat16)
    od_ref[...] = wd_ref[...].astype(jnp.bfloat16)


def _cast_experts(wg, wu, wd):
    nj = 4
    fj = _FF // nj
    return pl.pallas_call(
        _cast_body,
        grid=(_E, nj),
        in_specs=[
            pl.BlockSpec((1, _D, fj), lambda i, j: (i, 0, j)),
            pl.BlockSpec((1, _D, fj), lambda i, j: (i, 0, j)),
            pl.BlockSpec((1, fj, _D), lambda i, j: (i, j, 0)),
        ],
        out_specs=[
            pl.BlockSpec((1, _D, fj), lambda i, j: (i, 0, j)),
            pl.BlockSpec((1, _D, fj), lambda i, j: (i, 0, j)),
            pl.BlockSpec((1, fj, _D), lambda i, j: (i, j, 0)),
        ],
        out_shape=[
            jax.ShapeDtypeStruct((_E, _D, _FF), jnp.bfloat16),
            jax.ShapeDtypeStruct((_E, _D, _FF), jnp.bfloat16),
            jax.ShapeDtypeStruct((_E, _FF, _D), jnp.bfloat16),
        ],
    )(wg, wu, wd)


def _cast4_body(a_ref, b_ref, c_ref, d_ref, oa_ref, ob_ref, oc_ref, od_ref):
    oa_ref[...] = a_ref[...].astype(jnp.bfloat16)
    ob_ref[...] = b_ref[...].astype(jnp.bfloat16)
    oc_ref[...] = c_ref[...].astype(jnp.bfloat16)
    od_ref[...] = d_ref[...].astype(jnp.bfloat16)


def _cast_shared(flat, wsg, wsu, wsd):
    return pl.pallas_call(
        _cast4_body,
        out_shape=[
            jax.ShapeDtypeStruct(flat.shape, jnp.bfloat16),
            jax.ShapeDtypeStruct(wsg.shape, jnp.bfloat16),
            jax.ShapeDtypeStruct(wsu.shape, jnp.bfloat16),
            jax.ShapeDtypeStruct(wsd.shape, jnp.bfloat16),
        ],
    )(flat, wsg, wsu, wsd)


def _ffn_body(te_ref, tok_ref, tbl_ref, wg_ref, wu_ref, wd_ref, w_ref,
              out_ref):
    idx = tok_ref[0]
    oh = (idx == lax.broadcasted_iota(jnp.int32, (_TM, _T), 1))
    xb = jnp.dot(oh.astype(jnp.bfloat16), tbl_ref[...],
                 preferred_element_type=jnp.float32).astype(jnp.bfloat16)
    g = jnp.dot(xb, wg_ref[0], preferred_element_type=jnp.float32)
    u = jnp.dot(xb, wu_ref[0], preferred_element_type=jnp.float32)
    h = g * jax.nn.sigmoid(g) * u * w_ref[...]
    out_ref[...] = jnp.dot(h.astype(jnp.bfloat16), wd_ref[0],
                           preferred_element_type=jnp.float32)


def _ffn_grouped(tile_expert, tok_col, tbl16, wg, wu, wd, w_col):
    grid_spec = pltpu.PrefetchScalarGridSpec(
        num_scalar_prefetch=1,
        grid=(_NT,),
        in_specs=[
            pl.BlockSpec((1, _TM, 1), lambda i, te: (i, 0, 0)),
            pl.BlockSpec((_T, _D), lambda i, te: (0, 0)),
            pl.BlockSpec((1, _D, _FF), lambda i, te: (te[i], 0, 0)),
            pl.BlockSpec((1, _D, _FF), lambda i, te: (te[i], 0, 0)),
            pl.BlockSpec((1, _FF, _D), lambda i, te: (te[i], 0, 0)),
            pl.BlockSpec((_TM, 1), lambda i, te: (i, 0)),
        ],
        out_specs=pl.BlockSpec((_TM, _D), lambda i, te: (i, 0)),
    )
    return pl.pallas_call(
        _ffn_body, grid_spec=grid_spec,
        out_shape=jax.ShapeDtypeStruct((_PP, _D), jnp.float32),
    )(tile_expert, tok_col, tbl16, wg, wu, wd, w_col)


def _sc_gather(table, idx):
    rows, dd = table.shape
    (batch,) = idx.shape
    info = plsc.get_sparse_core_info()
    nw = info.num_cores * info.num_subcores
    assert batch % (8 * nw) == 0 and dd % info.num_lanes == 0
    bw = batch // nw
    mesh = plsc.VectorSubcoreMesh(core_axis_name="c", subcore_axis_name="s")

    @functools.partial(
        pl.kernel, mesh=mesh,
        out_type=jax.ShapeDtypeStruct((batch, dd), table.dtype),
        scratch_types=[
            pltpu.VMEM((bw,), jnp.int32),
            pltpu.VMEM((bw, dd), table.dtype),
            pltpu.SemaphoreType.DMA,
        ],
    )
    def k(table_hbm, idx_hbm, out_hbm, idx_v, rows_v, sem):
        wid = lax.axis_index("s") * info.num_cores + lax.axis_index("c")
        base = wid * bw
        pltpu.sync_copy(idx_hbm.at[pl.ds(base, bw)], idx_v)
        pltpu.async_copy(table_hbm.at[idx_v], rows_v, sem).wait()
        pltpu.sync_copy(rows_v, out_hbm.at[pl.ds(base, bw)])

    return k(table, idx)


def _shared_body(x_ref, wg_ref, wu_ref, wd_ref, yp0_ref, yp1_ref, out_ref):
    xb = x_ref[...]
    g = jnp.dot(xb, wg_ref[...], preferred_element_type=jnp.float32)
    u = jnp.dot(xb, wu_ref[...], preferred_element_type=jnp.float32)
    h = g * jax.nn.sigmoid(g) * u
    y = jnp.dot(h.astype(jnp.bfloat16), wd_ref[...],
                preferred_element_type=jnp.float32)
    out_ref[...] = y + yp0_ref[...] + yp1_ref[...]


def _shared(flat16, wsg, wsu, wsd, yp):
    bt = 256
    nb = _T // bt
    return pl.pallas_call(
        _shared_body,
        grid=(nb,),
        in_specs=[
            pl.BlockSpec((bt, _D), lambda i: (i, 0)),
            pl.BlockSpec((_D, _FF), lambda i: (0, 0)),
            pl.BlockSpec((_D, _FF), lambda i: (0, 0)),
            pl.BlockSpec((_FF, _D), lambda i: (0, 0)),
            pl.BlockSpec((bt, _D), lambda i: (i, 0)),
            pl.BlockSpec((bt, _D), lambda i, _nb=nb: (i + _nb, 0)),
        ],
        out_specs=pl.BlockSpec((bt, _D), lambda i: (i, 0)),
        out_shape=jax.ShapeDtypeStruct((_T, _D), jnp.float32),
    )(flat16, wsg, wsu, wsd, yp, yp)


def _routing_metadata(r):
    ids = r[:, :_K].astype(jnp.int32)
    w = r[:, _K:2 * _K]
    e_flat = ids.reshape(-1)
    onehot = (e_flat[:, None] == jnp.arange(_E, dtype=jnp.int32)[None, :])
    oh32 = onehot.astype(jnp.int32)
    csum = jnp.cumsum(oh32, axis=0)
    rank = jnp.sum(csum * oh32, axis=1) - 1
    counts = csum[-1]
    pcounts = ((counts + _TM - 1) // _TM) * _TM
    off = jnp.concatenate(
        [jnp.zeros((1,), jnp.int32), jnp.cumsum(pcounts).astype(jnp.int32)])
    dest = jnp.sum(off[None, :_E] * oh32, axis=1) + rank
    tok = jnp.arange(_T * _K, dtype=jnp.int32) // _K
    w_bits = lax.bitcast_convert_type(w.reshape(-1), jnp.int32)
    packed = jnp.zeros((_PP, 2), jnp.int32).at[dest].set(
        jnp.stack([tok, w_bits], axis=-1))
    sorted_tok = packed[:, 0]
    w_sorted = lax.bitcast_convert_type(packed[:, 1], jnp.float32)
    tile_expert = jnp.clip(
        jnp.searchsorted(off, jnp.arange(_NT, dtype=jnp.int32) * _TM,
                         side="right") - 1,
        0, _E - 1).astype(jnp.int32)
    dest_k = dest.reshape(_T, _K).T.reshape(-1)
    return dest_k, sorted_tok, w_sorted, tile_expert


def kernel(x, Wr, br, Wg, Wu, Wd, Wsg, Wsu, Wsd):
    b, s, d = x.shape
    flat = x.reshape(-1, d)
    wr_pad = jnp.zeros((_D, 128), jnp.float32).at[:, :_E].set(Wr)
    br_pad = jnp.full((1, 128), _NEG, jnp.float32).at[0, :_E].set(br)
    r = _router(flat, wr_pad, br_pad)
    dest_k, sorted_tok, w_sorted, tile_expert = _routing_metadata(r)

    wg16, wu16, wd16 = _cast_experts(Wg, Wu, Wd)
    flat16, wsg16, wsu16, wsd16 = _cast_shared(flat, Wsg, Wsu, Wsd)

    ys = _ffn_grouped(tile_expert, sorted_tok.reshape(_NT, _TM, 1), flat16,
                      wg16, wu16, wd16, w_sorted.reshape(_PP, 1))
    yp = _sc_gather(ys, dest_k)
    y = _shared(flat16, wsg16, wsu16, wsd16, yp)
    return y.reshape(b, s, d)
